# Initial kernel scaffold; baseline (speedup 1.0000x reference)
import jax
import jax.numpy as jnp
from jax.experimental import pallas as pl


def kernel(edge_index, drug_struc, drug_expr, se_struc, pair, W1, b1, g1, be1, W2, b2, g2, be2, Wg, att_l, att_r, bg, Wm1, bm1, gm1, bem1, Wm2, bm2, gm2, bem2, Wm3, bm3, gm3, bem3, Wm5, bm5):
    raise NotImplementedError("write your pallas kernel here")


# speedup vs baseline: 32.0619x; 32.0619x over previous
import functools

import jax
import jax.numpy as jnp
from jax import lax
from jax.experimental import pallas as pl
from jax.experimental.pallas import tpu as pltpu
from jax.experimental.pallas import tpu_sc as plsc

N_NODE = 10000
D = 128
E_TOT = 320000
B_PAIR = 16384
NC, NS, LANES = 2, 16, 16
NW = NC * NS
EPW = E_TOT // NW
CH = 80
NPH = 5
EPP = EPW // NPH
NCHP = EPP // CH
ROWS_PT = 632
N_PAD = ROWS_PT * NS
RPW = B_PAIR // NW
GCH = 128

_EPS_BN = 1e-5


def _bn_lrelu(z, g, be, slope=0.01):
    m = jnp.mean(z, axis=0, keepdims=True)
    v = jnp.mean((z - m) ** 2, axis=0, keepdims=True)
    y = (z - m) / jnp.sqrt(v + _EPS_BN) * g + be
    return jnp.where(y >= 0, y, slope * y)


def _gat_prep_body(de_ref, ss_ref, wg_ref, attl_ref, attr_ref,
                   xl_ref, xr_ref, al_ref, ar_ref):
    xl = jnp.dot(de_ref[...], wg_ref[...], preferred_element_type=jnp.float32)
    xr = jnp.dot(ss_ref[...], wg_ref[...], preferred_element_type=jnp.float32)
    xl_ref[...] = xl
    xr_ref[...] = xr
    al_ref[...] = jnp.dot(xl, attl_ref[...], preferred_element_type=jnp.float32)
    ar_ref[...] = jnp.dot(xr, attr_ref[...], preferred_element_type=jnp.float32)


def _drugin_body(ds_ref, w1_ref, b1_ref, g1_ref, be1_ref,
                 w2_ref, b2_ref, g2_ref, be2_ref, out_ref):
    z1 = jnp.dot(ds_ref[...], w1_ref[...], preferred_element_type=jnp.float32)
    h = _bn_lrelu(z1 + b1_ref[...], g1_ref[...], be1_ref[...])
    z2 = jnp.dot(h, w2_ref[...], preferred_element_type=jnp.float32)
    out_ref[...] = _bn_lrelu(z2 + b2_ref[...], g2_ref[...], be2_ref[...])


def _fse_body(msg_ref, den_ref, xr_ref, bg_ref, out_ref):
    msg = msg_ref[0] + msg_ref[1]
    den = den_ref[0] + den_ref[1]
    out = msg / (den + 1e-16) + xr_ref[...] + bg_ref[...]
    out_ref[...] = jax.nn.sigmoid(out)


def _interact_body(fd_ref, fs_ref, wm1a_ref, wm1b_ref, bm1_ref, gm1_ref,
                   bem1_ref, wm2_ref, bm2_ref, gm2_ref, bem2_ref,
                   wm3_ref, bm3_ref, gm3_ref, bem3_ref, wm5_ref, bm5_ref,
                   out_ref):
    z1 = (jnp.dot(fd_ref[...], wm1a_ref[...], preferred_element_type=jnp.float32)
          + jnp.dot(fs_ref[...], wm1b_ref[...], preferred_element_type=jnp.float32))
    e1 = _bn_lrelu(z1 + bm1_ref[...], gm1_ref[...], bem1_ref[...])
    z2 = jnp.dot(e1, wm2_ref[...], preferred_element_type=jnp.float32)
    e2 = _bn_lrelu(z2 + bm2_ref[...], gm2_ref[...], bem2_ref[...])
    z3 = jnp.dot(e2, wm3_ref[...], preferred_element_type=jnp.float32)
    e3 = _bn_lrelu(z3 + bm3_ref[...], gm3_ref[...], bem3_ref[...])
    z5 = jnp.dot(e3, wm5_ref[...], preferred_element_type=jnp.float32)
    out_ref[...] = jax.nn.sigmoid(z5 + bm5_ref[...])


_MESH = plsc.VectorSubcoreMesh(core_axis_name="c", subcore_axis_name="s",
                               num_cores=NC, num_subcores=NS)
_SC_PARAMS = pltpu.CompilerParams(needs_layout_passes=False)


@functools.partial(
    pl.kernel,
    out_type=[jax.ShapeDtypeStruct((NC, N_PAD, D), jnp.float32),
              jax.ShapeDtypeStruct((NC, N_PAD), jnp.float32)],
    mesh=_MESH,
    scratch_types=[
        pltpu.VMEM((EPP,), jnp.int32),
        pltpu.VMEM((EPP,), jnp.int32),
        pltpu.VMEM((N_NODE,), jnp.float32),
        pltpu.VMEM((N_NODE,), jnp.float32),
        pltpu.VMEM((CH,), jnp.float32),
        pltpu.VMEM((CH,), jnp.int32),
        pltpu.VMEM((CH, D), jnp.float32),
        pltpu.VMEM((CH, D), jnp.float32),
        pltpu.SemaphoreType.DMA((2,)),
        pltpu.VMEM_SHARED((N_PAD, D), jnp.float32),
        pltpu.VMEM_SHARED((N_PAD,), jnp.float32),
    ],
    compiler_params=_SC_PARAMS,
)
def _edge_kernel(src_hbm, dst_hbm, al_hbm, ar_hbm, xl_hbm, zrow_hbm, zden_hbm,
                 msg_hbm, den_hbm,
                 src_v, dst_v, al_v, ar_v, w_v, dsti_v, rows_a, rows_b, sems,
                 msg_s, den_s):
    cid = lax.axis_index("c")
    sid = lax.axis_index("s")
    wid = cid * NS + sid
    base_e = wid * EPW

    pltpu.sync_copy(al_hbm, al_v)
    pltpu.sync_copy(ar_hbm, ar_v)

    r0 = sid * ROWS_PT
    pltpu.sync_copy(zrow_hbm.at[pl.ds(r0, ROWS_PT)],
                    msg_s.at[pl.ds(r0, ROWS_PT)])

    @pl.when(sid == 0)
    def _():
        pltpu.sync_copy(zden_hbm, den_s)

    plsc.subcore_barrier()

    def issue(c, buf, i):
        pltpu.async_copy(xl_hbm.at[src_v.at[pl.ds(c * CH, CH)]], buf,
                         sems.at[i])

    def wait(buf, i):
        pltpu.make_async_copy(xl_hbm.at[src_v.at[pl.ds(0, CH)]], buf,
                              sems.at[i]).wait()

    def compute(c, buf):
        for k in range(CH // LANES):
            off = c * CH + k * LANES
            s16 = src_v[pl.ds(off, LANES)]
            d16 = dst_v[pl.ds(off, LANES)]
            a = plsc.load_gather(al_v, [s16]) + plsc.load_gather(ar_v, [d16])
            a = jnp.where(a >= 0, a, 0.2 * a)
            w_v[pl.ds(k * LANES, LANES)] = jnp.exp(a)
            dsti_v[pl.ds(k * LANES, LANES)] = d16

        def sbody(e, carry):
            wspl = plsc.load_gather(w_v, [jnp.full((LANES,), 0, jnp.int32) + e])
            for ccol in range(D // LANES):
                sl = pl.ds(ccol * LANES, LANES)
                buf[e, sl] = buf[e, sl] * wspl
            return carry

        lax.fori_loop(0, CH, sbody, 0)

        pltpu.sync_copy(buf, msg_s.at[dsti_v], add=True)
        pltpu.sync_copy(w_v, den_s.at[dsti_v], add=True)

    def pair_body(t, carry):
        c0 = 2 * t
        issue(c0 + 1, rows_b, 1)
        wait(rows_a, 0)
        compute(c0, rows_a)
        issue(c0 + 2, rows_a, 0)
        wait(rows_b, 1)
        compute(c0 + 1, rows_b)
        return carry

    for p in range(NPH):
        pltpu.sync_copy(src_hbm.at[pl.ds(base_e + p * EPP, EPP)], src_v)
        pltpu.sync_copy(dst_hbm.at[pl.ds(base_e + p * EPP, EPP)], dst_v)
        issue(0, rows_a, 0)
        lax.fori_loop(0, (NCHP - 1) // 2, pair_body, 0)
        wait(rows_a, 0)
        compute(NCHP - 1, rows_a)

    plsc.subcore_barrier()

    pltpu.sync_copy(msg_s.at[pl.ds(r0, ROWS_PT)],
                    msg_hbm.at[cid, pl.ds(r0, ROWS_PT)])

    @pl.when(sid == 0)
    def _():
        pltpu.sync_copy(den_s, den_hbm.at[cid])


@functools.partial(
    pl.kernel,
    out_type=[jax.ShapeDtypeStruct((B_PAIR, D), jnp.float32),
              jax.ShapeDtypeStruct((B_PAIR, D), jnp.float32)],
    mesh=_MESH,
    scratch_types=[
        pltpu.VMEM((RPW,), jnp.int32),
        pltpu.VMEM((RPW,), jnp.int32),
        pltpu.VMEM((GCH, D), jnp.float32),
        pltpu.VMEM((GCH, D), jnp.float32),
        pltpu.SemaphoreType.DMA((2,)),
    ],
    compiler_params=_SC_PARAMS,
)
def _pair_gather_kernel(fd_hbm, fs_hbm, p0_hbm, p1_hbm, g0_hbm, g1_hbm,
                        i0_v, i1_v, buf_a, buf_b, sems):
    cid = lax.axis_index("c")
    sid = lax.axis_index("s")
    wid = cid * NS + sid
    base = wid * RPW

    pltpu.sync_copy(p0_hbm.at[pl.ds(base, RPW)], i0_v)
    pltpu.sync_copy(p1_hbm.at[pl.ds(base, RPW)], i1_v)

    work = []
    for k in range(RPW // GCH):
        work.append((fd_hbm, i0_v, g0_hbm, k))
    for k in range(RPW // GCH):
        work.append((fs_hbm, i1_v, g1_hbm, k))

    bufs = (buf_a, buf_b)

    def issue(j):
        tbl, idx, _, k = work[j]
        pltpu.async_copy(tbl.at[idx.at[pl.ds(k * GCH, GCH)]], bufs[j % 2],
                         sems.at[j % 2])

    issue(0)
    for j in range(len(work)):
        if j + 1 < len(work):
            issue(j + 1)
        tbl, _, out, k = work[j]
        pltpu.make_async_copy(tbl.at[pl.ds(0, GCH)], bufs[j % 2],
                              sems.at[j % 2]).wait()
        pltpu.sync_copy(bufs[j % 2], out.at[pl.ds(base + k * GCH, GCH)])


def kernel(edge_index, drug_struc, drug_expr, se_struc, pair,
           W1, b1, g1, be1, W2, b2, g2, be2, Wg, att_l, att_r, bg,
           Wm1, bm1, gm1, bem1, Wm2, bm2, gm2, bem2, Wm3, bm3, gm3, bem3,
           Wm5, bm5):
    f32 = jnp.float32
    src = edge_index[0].astype(jnp.int32)
    dst = edge_index[1].astype(jnp.int32)
    p0 = pair[0].astype(jnp.int32)
    p1 = pair[1].astype(jnp.int32)

    row2 = lambda x: x.reshape(1, -1)

    xl, xr, al2, ar2 = pl.pallas_call(
        _gat_prep_body,
        out_shape=[jax.ShapeDtypeStruct((N_NODE, D), f32),
                   jax.ShapeDtypeStruct((N_NODE, D), f32),
                   jax.ShapeDtypeStruct((N_NODE, 1), f32),
                   jax.ShapeDtypeStruct((N_NODE, 1), f32)],
    )(drug_expr, se_struc, Wg, att_l.reshape(D, 1), att_r.reshape(D, 1))

    f_drug = pl.pallas_call(
        _drugin_body,
        out_shape=jax.ShapeDtypeStruct((N_NODE, D), f32),
    )(drug_struc, W1, row2(b1), row2(g1), row2(be1),
      W2, row2(b2), row2(g2), row2(be2))

    zrow = jnp.zeros((N_PAD, D), f32)
    zden = jnp.zeros((N_PAD,), f32)
    msg2, den2 = _edge_kernel(src, dst, al2.reshape(-1), ar2.reshape(-1),
                              xl, zrow, zden)

    f_se = pl.pallas_call(
        _fse_body,
        out_shape=jax.ShapeDtypeStruct((N_NODE, D), f32),
    )(msg2[:, :N_NODE], den2[:, :N_NODE].reshape(NC, N_NODE, 1), xr, row2(bg))

    fd_g, fs_g = _pair_gather_kernel(f_drug, f_se, p0, p1)

    adj = pl.pallas_call(
        _interact_body,
        out_shape=jax.ShapeDtypeStruct((B_PAIR, 1), f32),
    )(fd_g, fs_g, Wm1[:D], Wm1[D:], row2(bm1), row2(gm1), row2(bem1),
      Wm2, row2(bm2), row2(gm2), row2(bem2),
      Wm3, row2(bm3), row2(gm3), row2(bem3),
      Wm5, row2(bm5))

    return adj, f_se


# speedup vs baseline: 36.2966x; 1.1321x over previous
import functools

import jax
import jax.numpy as jnp
from jax import lax
from jax.experimental import pallas as pl
from jax.experimental.pallas import tpu as pltpu
from jax.experimental.pallas import tpu_sc as plsc

N_NODE = 10000
D = 128
E_TOT = 320000
B_PAIR = 16384
NC, NS, LANES = 2, 16, 16
NW = NC * NS
EPW = E_TOT // NW
CH = 80
NPH = 5
EPP = EPW // NPH
NCHP = EPP // CH
ROWS_PT = 632
N_PAD = ROWS_PT * NS
RPW = B_PAIR // NW
GCH = 128

_EPS_BN = 1e-5


def _bn_lrelu(z, g, be, slope=0.01):
    m = jnp.mean(z, axis=0, keepdims=True)
    v = jnp.mean((z - m) ** 2, axis=0, keepdims=True)
    y = (z - m) / jnp.sqrt(v + _EPS_BN) * g + be
    return jnp.where(y >= 0, y, slope * y)


def _gat_prep_body(de_ref, ss_ref, wg_ref, attl_ref, attr_ref,
                   xl_ref, xr_ref, al_ref, ar_ref):
    xl = jnp.dot(de_ref[...], wg_ref[...], preferred_element_type=jnp.float32)
    xr = jnp.dot(ss_ref[...], wg_ref[...], preferred_element_type=jnp.float32)
    xl_ref[...] = xl
    xr_ref[...] = xr
    al_ref[...] = jnp.dot(xl, attl_ref[...], preferred_element_type=jnp.float32)
    ar_ref[...] = jnp.dot(xr, attr_ref[...], preferred_element_type=jnp.float32)


def _drugin_body(ds_ref, w1_ref, b1_ref, g1_ref, be1_ref,
                 w2_ref, b2_ref, g2_ref, be2_ref, out_ref):
    z1 = jnp.dot(ds_ref[...], w1_ref[...], preferred_element_type=jnp.float32)
    h = _bn_lrelu(z1 + b1_ref[...], g1_ref[...], be1_ref[...])
    z2 = jnp.dot(h, w2_ref[...], preferred_element_type=jnp.float32)
    out_ref[...] = _bn_lrelu(z2 + b2_ref[...], g2_ref[...], be2_ref[...])


def _fse_body(msg_ref, den_ref, xr_ref, bg_ref, out_ref):
    msg = msg_ref[0] + msg_ref[1]
    den = den_ref[0] + den_ref[1]
    out = msg / (den + 1e-16) + xr_ref[...] + bg_ref[...]
    out_ref[...] = jax.nn.sigmoid(out)


def _interact_body(fd_ref, fs_ref, wm1a_ref, wm1b_ref, bm1_ref, gm1_ref,
                   bem1_ref, wm2_ref, bm2_ref, gm2_ref, bem2_ref,
                   wm3_ref, bm3_ref, gm3_ref, bem3_ref, wm5_ref, bm5_ref,
                   out_ref):
    z1 = (jnp.dot(fd_ref[...], wm1a_ref[...], preferred_element_type=jnp.float32)
          + jnp.dot(fs_ref[...], wm1b_ref[...], preferred_element_type=jnp.float32))
    e1 = _bn_lrelu(z1 + bm1_ref[...], gm1_ref[...], bem1_ref[...])
    z2 = jnp.dot(e1, wm2_ref[...], preferred_element_type=jnp.float32)
    e2 = _bn_lrelu(z2 + bm2_ref[...], gm2_ref[...], bem2_ref[...])
    z3 = jnp.dot(e2, wm3_ref[...], preferred_element_type=jnp.float32)
    e3 = _bn_lrelu(z3 + bm3_ref[...], gm3_ref[...], bem3_ref[...])
    z5 = jnp.dot(e3, wm5_ref[...], preferred_element_type=jnp.float32)
    out_ref[...] = jax.nn.sigmoid(z5 + bm5_ref[...])


_MESH = plsc.VectorSubcoreMesh(core_axis_name="c", subcore_axis_name="s",
                               num_cores=NC, num_subcores=NS)
_SC_PARAMS = pltpu.CompilerParams(needs_layout_passes=False)


@functools.partial(
    pl.kernel,
    out_type=[jax.ShapeDtypeStruct((NC, N_PAD, D), jnp.float32),
              jax.ShapeDtypeStruct((NC, N_PAD), jnp.float32)],
    mesh=_MESH,
    scratch_types=[
        pltpu.VMEM((EPP,), jnp.int32),
        pltpu.VMEM((EPP,), jnp.int32),
        pltpu.VMEM((N_NODE,), jnp.float32),
        pltpu.VMEM((N_NODE,), jnp.float32),
        pltpu.VMEM((CH,), jnp.float32),
        pltpu.VMEM((CH,), jnp.int32),
        pltpu.VMEM((CH, D), jnp.float32),
        pltpu.VMEM((CH, D), jnp.float32),
        pltpu.SemaphoreType.DMA((2,)),
        pltpu.VMEM_SHARED((N_PAD, D), jnp.float32),
        pltpu.VMEM_SHARED((N_PAD,), jnp.float32),
    ],
    compiler_params=_SC_PARAMS,
)
def _edge_kernel(src_hbm, dst_hbm, al_hbm, ar_hbm, xl_hbm, zrow_hbm, zden_hbm,
                 msg_hbm, den_hbm,
                 src_v, dst_v, al_v, ar_v, w_v, dsti_v, rows_a, rows_b, sems,
                 msg_s, den_s):
    cid = lax.axis_index("c")
    sid = lax.axis_index("s")
    wid = cid * NS + sid
    base_e = wid * EPW

    pltpu.sync_copy(al_hbm, al_v)
    pltpu.sync_copy(ar_hbm, ar_v)

    r0 = sid * ROWS_PT
    pltpu.sync_copy(zrow_hbm.at[pl.ds(r0, ROWS_PT)],
                    msg_s.at[pl.ds(r0, ROWS_PT)])

    @pl.when(sid == 0)
    def _():
        pltpu.sync_copy(zden_hbm, den_s)

    plsc.subcore_barrier()

    def issue(c, buf, i):
        pltpu.async_copy(xl_hbm.at[src_v.at[pl.ds(c * CH, CH)]], buf,
                         sems.at[i])

    def wait(buf, i):
        pltpu.make_async_copy(xl_hbm.at[src_v.at[pl.ds(0, CH)]], buf,
                              sems.at[i]).wait()

    def compute(c, buf):
        for k in range(CH // LANES):
            off = c * CH + k * LANES
            s16 = src_v[pl.ds(off, LANES)]
            d16 = dst_v[pl.ds(off, LANES)]
            a = plsc.load_gather(al_v, [s16]) + plsc.load_gather(ar_v, [d16])
            a = jnp.where(a >= 0, a, 0.2 * a)
            w_v[pl.ds(k * LANES, LANES)] = jnp.exp(a)
            dsti_v[pl.ds(k * LANES, LANES)] = d16

        @plsc.parallel_loop(0, CH, unroll=8)
        def _scale(e):
            wspl = plsc.load_gather(w_v, [jnp.full((LANES,), 0, jnp.int32) + e])
            for ccol in range(D // LANES):
                sl = pl.ds(ccol * LANES, LANES)
                buf[e, sl] = buf[e, sl] * wspl

        pltpu.sync_copy(buf, msg_s.at[dsti_v], add=True)
        pltpu.sync_copy(w_v, den_s.at[dsti_v], add=True)

    def pair_body(t, carry):
        c0 = 2 * t
        issue(c0 + 1, rows_b, 1)
        wait(rows_a, 0)
        compute(c0, rows_a)
        issue(c0 + 2, rows_a, 0)
        wait(rows_b, 1)
        compute(c0 + 1, rows_b)
        return carry

    for p in range(NPH):
        pltpu.sync_copy(src_hbm.at[pl.ds(base_e + p * EPP, EPP)], src_v)
        pltpu.sync_copy(dst_hbm.at[pl.ds(base_e + p * EPP, EPP)], dst_v)
        issue(0, rows_a, 0)
        lax.fori_loop(0, (NCHP - 1) // 2, pair_body, 0)
        wait(rows_a, 0)
        compute(NCHP - 1, rows_a)

    plsc.subcore_barrier()

    pltpu.sync_copy(msg_s.at[pl.ds(r0, ROWS_PT)],
                    msg_hbm.at[cid, pl.ds(r0, ROWS_PT)])

    @pl.when(sid == 0)
    def _():
        pltpu.sync_copy(den_s, den_hbm.at[cid])


@functools.partial(
    pl.kernel,
    out_type=[jax.ShapeDtypeStruct((B_PAIR, D), jnp.float32),
              jax.ShapeDtypeStruct((B_PAIR, D), jnp.float32)],
    mesh=_MESH,
    scratch_types=[
        pltpu.VMEM((RPW,), jnp.int32),
        pltpu.VMEM((RPW,), jnp.int32),
        pltpu.VMEM((GCH, D), jnp.float32),
        pltpu.VMEM((GCH, D), jnp.float32),
        pltpu.SemaphoreType.DMA((2,)),
    ],
    compiler_params=_SC_PARAMS,
)
def _pair_gather_kernel(fd_hbm, fs_hbm, p0_hbm, p1_hbm, g0_hbm, g1_hbm,
                        i0_v, i1_v, buf_a, buf_b, sems):
    cid = lax.axis_index("c")
    sid = lax.axis_index("s")
    wid = cid * NS + sid
    base = wid * RPW

    pltpu.sync_copy(p0_hbm.at[pl.ds(base, RPW)], i0_v)
    pltpu.sync_copy(p1_hbm.at[pl.ds(base, RPW)], i1_v)

    work = []
    for k in range(RPW // GCH):
        work.append((fd_hbm, i0_v, g0_hbm, k))
    for k in range(RPW // GCH):
        work.append((fs_hbm, i1_v, g1_hbm, k))

    bufs = (buf_a, buf_b)

    def issue(j):
        tbl, idx, _, k = work[j]
        pltpu.async_copy(tbl.at[idx.at[pl.ds(k * GCH, GCH)]], bufs[j % 2],
                         sems.at[j % 2])

    issue(0)
    for j in range(len(work)):
        if j + 1 < len(work):
            issue(j + 1)
        tbl, _, out, k = work[j]
        pltpu.make_async_copy(tbl.at[pl.ds(0, GCH)], bufs[j % 2],
                              sems.at[j % 2]).wait()
        pltpu.sync_copy(bufs[j % 2], out.at[pl.ds(base + k * GCH, GCH)])


def kernel(edge_index, drug_struc, drug_expr, se_struc, pair,
           W1, b1, g1, be1, W2, b2, g2, be2, Wg, att_l, att_r, bg,
           Wm1, bm1, gm1, bem1, Wm2, bm2, gm2, bem2, Wm3, bm3, gm3, bem3,
           Wm5, bm5):
    f32 = jnp.float32
    src = edge_index[0].astype(jnp.int32)
    dst = edge_index[1].astype(jnp.int32)
    p0 = pair[0].astype(jnp.int32)
    p1 = pair[1].astype(jnp.int32)

    row2 = lambda x: x.reshape(1, -1)

    xl, xr, al2, ar2 = pl.pallas_call(
        _gat_prep_body,
        out_shape=[jax.ShapeDtypeStruct((N_NODE, D), f32),
                   jax.ShapeDtypeStruct((N_NODE, D), f32),
                   jax.ShapeDtypeStruct((N_NODE, 1), f32),
                   jax.ShapeDtypeStruct((N_NODE, 1), f32)],
    )(drug_expr, se_struc, Wg, att_l.reshape(D, 1), att_r.reshape(D, 1))

    f_drug = pl.pallas_call(
        _drugin_body,
        out_shape=jax.ShapeDtypeStruct((N_NODE, D), f32),
    )(drug_struc, W1, row2(b1), row2(g1), row2(be1),
      W2, row2(b2), row2(g2), row2(be2))

    zrow = jnp.zeros((N_PAD, D), f32)
    zden = jnp.zeros((N_PAD,), f32)
    msg2, den2 = _edge_kernel(src, dst, al2.reshape(-1), ar2.reshape(-1),
                              xl, zrow, zden)

    f_se = pl.pallas_call(
        _fse_body,
        out_shape=jax.ShapeDtypeStruct((N_NODE, D), f32),
    )(msg2[:, :N_NODE], den2[:, :N_NODE].reshape(NC, N_NODE, 1), xr, row2(bg))

    fd_g, fs_g = _pair_gather_kernel(f_drug, f_se, p0, p1)

    adj = pl.pallas_call(
        _interact_body,
        out_shape=jax.ShapeDtypeStruct((B_PAIR, 1), f32),
    )(fd_g, fs_g, Wm1[:D], Wm1[D:], row2(bm1), row2(gm1), row2(bem1),
      Wm2, row2(bm2), row2(gm2), row2(bem2),
      Wm3, row2(bm3), row2(gm3), row2(bem3),
      Wm5, row2(bm5))

    return adj, f_se


# speedup vs baseline: 38.2294x; 1.0533x over previous
import functools

import jax
import jax.numpy as jnp
from jax import lax
from jax.experimental import pallas as pl
from jax.experimental.pallas import tpu as pltpu
from jax.experimental.pallas import tpu_sc as plsc

N_NODE = 10000
D = 128
E_TOT = 320000
B_PAIR = 16384
NC, NS, LANES = 2, 16, 16
NW = NC * NS
EPW = E_TOT // NW
CH = 80
NPH = 5
EPP = EPW // NPH
NCHP = EPP // CH
ROWS_PT = 632
N_PAD = ROWS_PT * NS
RPW = B_PAIR // NW
GCH = 128

_EPS_BN = 1e-5


def _bn_lrelu(z, g, be, slope=0.01):
    m = jnp.mean(z, axis=0, keepdims=True)
    v = jnp.mean((z - m) ** 2, axis=0, keepdims=True)
    y = (z - m) / jnp.sqrt(v + _EPS_BN) * g + be
    return jnp.where(y >= 0, y, slope * y)


def _gat_prep_body(de_ref, ss_ref, wg_ref, attl_ref, attr_ref,
                   xl_ref, xr_ref, al_ref, ar_ref):
    xl = jnp.dot(de_ref[...], wg_ref[...], preferred_element_type=jnp.float32)
    xr = jnp.dot(ss_ref[...], wg_ref[...], preferred_element_type=jnp.float32)
    xl_ref[...] = xl
    xr_ref[...] = xr
    al_ref[...] = jnp.dot(xl, attl_ref[...], preferred_element_type=jnp.float32)
    ar_ref[...] = jnp.dot(xr, attr_ref[...], preferred_element_type=jnp.float32)


def _drugin_body(ds_ref, w1_ref, b1_ref, g1_ref, be1_ref,
                 w2_ref, b2_ref, g2_ref, be2_ref, out_ref):
    z1 = jnp.dot(ds_ref[...], w1_ref[...], preferred_element_type=jnp.float32)
    h = _bn_lrelu(z1 + b1_ref[...], g1_ref[...], be1_ref[...])
    z2 = jnp.dot(h, w2_ref[...], preferred_element_type=jnp.float32)
    out_ref[...] = _bn_lrelu(z2 + b2_ref[...], g2_ref[...], be2_ref[...])


def _fse_body(msg_ref, den_ref, xr_ref, bg_ref, out_ref):
    msg = msg_ref[0] + msg_ref[1]
    den = den_ref[0] + den_ref[1]
    out = msg / (den + 1e-16) + xr_ref[...] + bg_ref[...]
    out_ref[...] = jax.nn.sigmoid(out)


def _interact_body(fd_ref, fs_ref, wm1a_ref, wm1b_ref, bm1_ref, gm1_ref,
                   bem1_ref, wm2_ref, bm2_ref, gm2_ref, bem2_ref,
                   wm3_ref, bm3_ref, gm3_ref, bem3_ref, wm5_ref, bm5_ref,
                   out_ref):
    z1 = (jnp.dot(fd_ref[...], wm1a_ref[...], preferred_element_type=jnp.float32)
          + jnp.dot(fs_ref[...], wm1b_ref[...], preferred_element_type=jnp.float32))
    e1 = _bn_lrelu(z1 + bm1_ref[...], gm1_ref[...], bem1_ref[...])
    z2 = jnp.dot(e1, wm2_ref[...], preferred_element_type=jnp.float32)
    e2 = _bn_lrelu(z2 + bm2_ref[...], gm2_ref[...], bem2_ref[...])
    z3 = jnp.dot(e2, wm3_ref[...], preferred_element_type=jnp.float32)
    e3 = _bn_lrelu(z3 + bm3_ref[...], gm3_ref[...], bem3_ref[...])
    z5 = jnp.dot(e3, wm5_ref[...], preferred_element_type=jnp.float32)
    out_ref[...] = jax.nn.sigmoid(z5 + bm5_ref[...])


_MESH = plsc.VectorSubcoreMesh(core_axis_name="c", subcore_axis_name="s",
                               num_cores=NC, num_subcores=NS)
_SC_PARAMS = pltpu.CompilerParams(needs_layout_passes=False)


@functools.partial(
    pl.kernel,
    out_type=jax.ShapeDtypeStruct((E_TOT,), jnp.float32),
    mesh=_MESH,
    scratch_types=[
        pltpu.VMEM((EPP,), jnp.int32),
        pltpu.VMEM((EPP,), jnp.int32),
        pltpu.VMEM((N_NODE,), jnp.float32),
        pltpu.VMEM((N_NODE,), jnp.float32),
        pltpu.VMEM((EPP,), jnp.float32),
    ],
    compiler_params=_SC_PARAMS,
)
def _wden_kernel(src_hbm, dst_hbm, al_hbm, ar_hbm,
                 w_hbm,
                 src_v, dst_v, al_v, ar_v, w_v):
    cid = lax.axis_index("c")
    sid = lax.axis_index("s")
    base_e = (cid * NS + sid) * EPW

    pltpu.sync_copy(al_hbm, al_v)
    pltpu.sync_copy(ar_hbm, ar_v)

    for p in range(NPH):
        pltpu.sync_copy(src_hbm.at[pl.ds(base_e + p * EPP, EPP)], src_v)
        pltpu.sync_copy(dst_hbm.at[pl.ds(base_e + p * EPP, EPP)], dst_v)

        def wbody(k, carry):
            off = k * LANES
            s16 = src_v[pl.ds(off, LANES)]
            d16 = dst_v[pl.ds(off, LANES)]
            a = plsc.load_gather(al_v, [s16]) + plsc.load_gather(ar_v, [d16])
            a = jnp.where(a >= 0, a, 0.2 * a)
            w_v[pl.ds(off, LANES)] = jnp.exp(a)
            return carry

        lax.fori_loop(0, EPP // LANES, wbody, 0)
        pltpu.sync_copy(w_v, w_hbm.at[pl.ds(base_e + p * EPP, EPP)])


MCH = 120
NFC = EPW // MCH
MT = EPW - NFC * MCH


@functools.partial(
    pl.kernel,
    out_type=[jax.ShapeDtypeStruct((NC, N_PAD, D), jnp.float32),
              jax.ShapeDtypeStruct((NC, N_PAD), jnp.float32)],
    mesh=_MESH,
    scratch_types=[
        [pltpu.VMEM((MCH,), jnp.int32) for _ in range(3)],
        [pltpu.VMEM((MCH,), jnp.int32) for _ in range(3)],
        [pltpu.VMEM((MCH,), jnp.float32) for _ in range(3)],
        [pltpu.VMEM((MCH, D), jnp.float32) for _ in range(3)],
        [pltpu.VMEM((MCH,), jnp.int32) for _ in range(2)],
        [pltpu.VMEM((MCH,), jnp.float32) for _ in range(2)],
        pltpu.VMEM((MT,), jnp.int32),
        pltpu.VMEM((MT,), jnp.int32),
        pltpu.VMEM((MT,), jnp.float32),
        pltpu.SemaphoreType.DMA((3,)),
        pltpu.SemaphoreType.DMA((3,)),
        pltpu.SemaphoreType.DMA((3,)),
        pltpu.VMEM_SHARED((N_PAD, D), jnp.float32),
        pltpu.VMEM_SHARED((N_PAD,), jnp.float32),
    ],
    compiler_params=_SC_PARAMS,
)
def _msg_kernel(src_hbm, dst_hbm, w_hbm, xl_hbm, zrow_hbm, zden_hbm,
                msg_hbm, den_hbm,
                isrc, idst, wb, rows, sd, sw, tsrc, tdst, tw,
                semi, semg, sems, msg_s, den_s):
    cid = lax.axis_index("c")
    sid = lax.axis_index("s")
    base_e = (cid * NS + sid) * EPW

    r0 = sid * ROWS_PT
    pltpu.sync_copy(zrow_hbm.at[pl.ds(r0, ROWS_PT)],
                    msg_s.at[pl.ds(r0, ROWS_PT)])

    @pl.when(sid == 0)
    def _():
        pltpu.sync_copy(zden_hbm, den_s)

    plsc.subcore_barrier()

    def stage_idx(c, i):
        b = base_e + c * MCH
        pltpu.async_copy(src_hbm.at[pl.ds(b, MCH)], isrc[i], semi.at[i])
        pltpu.async_copy(dst_hbm.at[pl.ds(b, MCH)], idst[i], semi.at[i])
        pltpu.async_copy(w_hbm.at[pl.ds(b, MCH)], wb[i], semi.at[i])

    def wait_idx(i):
        pltpu.make_async_copy(src_hbm.at[pl.ds(0, MCH)], isrc[i],
                              semi.at[i]).wait()
        pltpu.make_async_copy(dst_hbm.at[pl.ds(0, MCH)], idst[i],
                              semi.at[i]).wait()
        pltpu.make_async_copy(w_hbm.at[pl.ds(0, MCH)], wb[i],
                              semi.at[i]).wait()

    def issue_gather(i):
        pltpu.async_copy(xl_hbm.at[isrc[i]], rows[i], semg.at[i])

    def wait_gather(i):
        pltpu.make_async_copy(xl_hbm.at[pl.ds(0, MCH)], rows[i],
                              semg.at[i]).wait()

    def scale_and_stage(i, j):
        buf = rows[i]
        wv = wb[i]

        @plsc.parallel_loop(0, MCH, unroll=8)
        def _scale(e):
            wspl = plsc.load_gather(wv, [jnp.full((LANES,), 0, jnp.int32) + e])
            for ccol in range(D // LANES):
                sl = pl.ds(ccol * LANES, LANES)
                buf[e, sl] = buf[e, sl] * wspl

        for k in range(MCH // LANES):
            sd[j][pl.ds(k * LANES, LANES)] = idst[i][pl.ds(k * LANES, LANES)]
            sw[j][pl.ds(k * LANES, LANES)] = wv[pl.ds(k * LANES, LANES)]
        sd[j][pl.ds(MCH - LANES, LANES)] = idst[i][pl.ds(MCH - LANES, LANES)]
        sw[j][pl.ds(MCH - LANES, LANES)] = wv[pl.ds(MCH - LANES, LANES)]

    def issue_scatter(i, j):
        pltpu.async_copy(rows[i], msg_s.at[sd[j]], sems.at[i], add=True)
        pltpu.async_copy(sw[j], den_s.at[sd[j]], sems.at[i], add=True)

    def wait_scatter(i, j):
        pltpu.make_async_copy(zrow_hbm.at[pl.ds(0, MCH)], rows[i],
                              sems.at[i]).wait()
        pltpu.make_async_copy(w_hbm.at[pl.ds(0, MCH)], sw[j],
                              sems.at[i]).wait()

    def step(c, i0, i1, i2, j, first=False, nxt=True, stage=True):
        if not first:
            wait_scatter(i1, j)
        if nxt:
            wait_idx(i1)
            issue_gather(i1)
        if stage:
            stage_idx(c + 2, i2)
        wait_gather(i0)
        scale_and_stage(i0, j)
        issue_scatter(i0, j)

    stage_idx(0, 0)
    stage_idx(1, 1)
    wait_idx(0)
    issue_gather(0)
    step(0, 0, 1, 2, 0, first=True)
    step(1, 1, 2, 0, 1, first=True)

    def loop_body(t, carry):
        c = 2 + 6 * t
        step(c, 2, 0, 1, 0)
        step(c + 1, 0, 1, 2, 1)
        step(c + 2, 1, 2, 0, 0)
        step(c + 3, 2, 0, 1, 1)
        step(c + 4, 0, 1, 2, 0)
        step(c + 5, 1, 2, 0, 1)
        return carry

    lax.fori_loop(0, 13, loop_body, 0)
    step(80, 2, 0, 1, 0)
    step(81, 0, 1, 2, 1, stage=False)
    step(82, 1, 2, 0, 0, nxt=False, stage=False)
    wait_scatter(0, 1)
    wait_scatter(1, 0)

    bt = base_e + NFC * MCH
    pltpu.sync_copy(src_hbm.at[pl.ds(bt, MT)], tsrc)
    pltpu.sync_copy(dst_hbm.at[pl.ds(bt, MT)], tdst)
    pltpu.sync_copy(w_hbm.at[pl.ds(bt, MT)], tw)
    pltpu.async_copy(xl_hbm.at[tsrc], rows[0].at[pl.ds(0, MT)],
                     semg.at[0]).wait()

    @plsc.parallel_loop(0, MT, unroll=8)
    def _scale_t(e):
        wspl = plsc.load_gather(tw, [jnp.full((LANES,), 0, jnp.int32) + e])
        for ccol in range(D // LANES):
            sl = pl.ds(ccol * LANES, LANES)
            rows[0][e, sl] = rows[0][e, sl] * wspl

    pltpu.sync_copy(rows[0].at[pl.ds(0, MT)], msg_s.at[tdst], add=True)
    pltpu.sync_copy(tw, den_s.at[tdst], add=True)

    plsc.subcore_barrier()
    pltpu.sync_copy(msg_s.at[pl.ds(r0, ROWS_PT)],
                    msg_hbm.at[cid, pl.ds(r0, ROWS_PT)])

    @pl.when(sid == 0)
    def _():
        pltpu.sync_copy(den_s, den_hbm.at[cid])


@functools.partial(
    pl.kernel,
    out_type=[jax.ShapeDtypeStruct((B_PAIR, D), jnp.float32),
              jax.ShapeDtypeStruct((B_PAIR, D), jnp.float32)],
    mesh=_MESH,
    scratch_types=[
        pltpu.VMEM((RPW,), jnp.int32),
        pltpu.VMEM((RPW,), jnp.int32),
        pltpu.VMEM((GCH, D), jnp.float32),
        pltpu.VMEM((GCH, D), jnp.float32),
        pltpu.SemaphoreType.DMA((2,)),
    ],
    compiler_params=_SC_PARAMS,
)
def _pair_gather_kernel(fd_hbm, fs_hbm, p0_hbm, p1_hbm, g0_hbm, g1_hbm,
                        i0_v, i1_v, buf_a, buf_b, sems):
    cid = lax.axis_index("c")
    sid = lax.axis_index("s")
    wid = cid * NS + sid
    base = wid * RPW

    pltpu.sync_copy(p0_hbm.at[pl.ds(base, RPW)], i0_v)
    pltpu.sync_copy(p1_hbm.at[pl.ds(base, RPW)], i1_v)

    work = []
    for k in range(RPW // GCH):
        work.append((fd_hbm, i0_v, g0_hbm, k))
    for k in range(RPW // GCH):
        work.append((fs_hbm, i1_v, g1_hbm, k))

    bufs = (buf_a, buf_b)

    def issue(j):
        tbl, idx, _, k = work[j]
        pltpu.async_copy(tbl.at[idx.at[pl.ds(k * GCH, GCH)]], bufs[j % 2],
                         sems.at[j % 2])

    issue(0)
    for j in range(len(work)):
        if j + 1 < len(work):
            issue(j + 1)
        tbl, _, out, k = work[j]
        pltpu.make_async_copy(tbl.at[pl.ds(0, GCH)], bufs[j % 2],
                              sems.at[j % 2]).wait()
        pltpu.sync_copy(bufs[j % 2], out.at[pl.ds(base + k * GCH, GCH)])


def kernel(edge_index, drug_struc, drug_expr, se_struc, pair,
           W1, b1, g1, be1, W2, b2, g2, be2, Wg, att_l, att_r, bg,
           Wm1, bm1, gm1, bem1, Wm2, bm2, gm2, bem2, Wm3, bm3, gm3, bem3,
           Wm5, bm5):
    f32 = jnp.float32
    src = edge_index[0].astype(jnp.int32)
    dst = edge_index[1].astype(jnp.int32)
    p0 = pair[0].astype(jnp.int32)
    p1 = pair[1].astype(jnp.int32)

    row2 = lambda x: x.reshape(1, -1)

    xl, xr, al2, ar2 = pl.pallas_call(
        _gat_prep_body,
        out_shape=[jax.ShapeDtypeStruct((N_NODE, D), f32),
                   jax.ShapeDtypeStruct((N_NODE, D), f32),
                   jax.ShapeDtypeStruct((N_NODE, 1), f32),
                   jax.ShapeDtypeStruct((N_NODE, 1), f32)],
    )(drug_expr, se_struc, Wg, att_l.reshape(D, 1), att_r.reshape(D, 1))

    f_drug = pl.pallas_call(
        _drugin_body,
        out_shape=jax.ShapeDtypeStruct((N_NODE, D), f32),
    )(drug_struc, W1, row2(b1), row2(g1), row2(be1),
      W2, row2(b2), row2(g2), row2(be2))

    zrow = jnp.zeros((N_PAD, D), f32)
    zden = jnp.zeros((N_PAD,), f32)
    w_e = _wden_kernel(src, dst, al2.reshape(-1), ar2.reshape(-1))
    msg2, denP = _msg_kernel(src, dst, w_e, xl, zrow, zden)

    f_se = pl.pallas_call(
        _fse_body,
        out_shape=jax.ShapeDtypeStruct((N_NODE, D), f32),
    )(msg2[:, :N_NODE], denP[:, :N_NODE].reshape(NC, N_NODE, 1), xr, row2(bg))

    fd_g, fs_g = _pair_gather_kernel(f_drug, f_se, p0, p1)

    adj = pl.pallas_call(
        _interact_body,
        out_shape=jax.ShapeDtypeStruct((B_PAIR, 1), f32),
    )(fd_g, fs_g, Wm1[:D], Wm1[D:], row2(bm1), row2(gm1), row2(bem1),
      Wm2, row2(bm2), row2(gm2), row2(bem2),
      Wm3, row2(bm3), row2(gm3), row2(bem3),
      Wm5, row2(bm5))

    return adj, f_se
